# baseline (speedup 1.0000x reference)
import jax
import jax.numpy as jnp
from jax.experimental import pallas as pl
from jax.experimental.pallas import tpu as pltpu

_TOKENS = 8192
_DIM = 7168
_NE = 384
_K = 6
_SCALE = 2.5
_BT = 512
_NSUB = 2
_BS = _BT // _NSUB


def _topk_rows(s, biased):
    iota = jax.lax.broadcasted_iota(
        jnp.int32, (_BS, _NE), 1).astype(jnp.float32)
    cur = biased
    vals, idxs = [], []
    for j in range(_K):
        m = jnp.max(cur, axis=1, keepdims=True)
        idx = jnp.min(jnp.where(cur == m, iota, jnp.float32(_NE)), axis=1)
        sel = iota == idx[:, None]
        vals.append(jnp.sum(jnp.where(sel, s, 0.0), axis=1))
        idxs.append(idx)
        if j + 1 < _K:
            cur = jnp.where(sel, -jnp.inf, cur)
    return idxs, vals


def _gate_body(x_ref, w_ref, bias_ref, w_out_ref, i_out_ref, wt_ref):
    @pl.when(pl.program_id(0) == 0)
    def _():
        wt_ref[...] = w_ref[...].T

    wt = wt_ref[...]
    bias = bias_ref[...]
    scs = []
    for h in range(_NSUB):
        x = x_ref[h * _BS:(h + 1) * _BS, :]
        scs.append(jax.lax.dot_general(
            x, wt, (((1,), (0,)), ((), ())),
            preferred_element_type=jnp.float32))
    for h in range(_NSUB):
        s = jnp.sqrt(jax.nn.softplus(scs[h]))
        idxs, vals = _topk_rows(s, s + bias)
        inv = _SCALE / (vals[0] + vals[1] + vals[2]
                        + vals[3] + vals[4] + vals[5])
        col = pl.ds(h * _BS, _BS)
        for j in range(_K):
            i_out_ref[j, col] = idxs[j]
            w_out_ref[j, col] = vals[j] * inv


def kernel(x, weight, bias):
    bias2 = bias.reshape(1, _NE)
    w_out, i_out = pl.pallas_call(
        _gate_body,
        grid=(_TOKENS // _BT,),
        in_specs=[
            pl.BlockSpec((_BT, _DIM), lambda i: (i, 0)),
            pl.BlockSpec((_NE, _DIM), lambda i: (0, 0)),
            pl.BlockSpec((1, _NE), lambda i: (0, 0)),
        ],
        out_specs=[
            pl.BlockSpec((8, _BT), lambda i: (0, i)),
            pl.BlockSpec((8, _BT), lambda i: (0, i)),
        ],
        out_shape=[
            jax.ShapeDtypeStruct((8, _TOKENS), jnp.float32),
            jax.ShapeDtypeStruct((8, _TOKENS), jnp.float32),
        ],
        scratch_shapes=[pltpu.VMEM((_DIM, _NE), jnp.float32)],
    )(x, weight, bias2)
    return w_out[:_K].T, i_out[:_K].T.astype(jnp.int32)


# speedup vs baseline: 1.3376x; 1.3376x over previous
import jax
import jax.numpy as jnp
from jax.experimental import pallas as pl
from jax.experimental.pallas import tpu as pltpu

_TOKENS = 8192
_DIM = 7168
_NE = 384
_K = 6
_SCALE = 2.5
_BT = 256
_NSUB = 2
_BS = _BT // _NSUB


def _topk_rows(s, biased):
    iota = jax.lax.broadcasted_iota(
        jnp.int32, (_BS, _NE), 1).astype(jnp.float32)
    cur = biased
    vals, idxs = [], []
    for j in range(_K):
        m = jnp.max(cur, axis=1, keepdims=True)
        idx = jnp.min(jnp.where(cur == m, iota, jnp.float32(_NE)), axis=1)
        sel = iota == idx[:, None]
        vals.append(jnp.sum(jnp.where(sel, s, 0.0), axis=1))
        idxs.append(idx)
        if j + 1 < _K:
            cur = jnp.where(sel, -jnp.inf, cur)
    return idxs, vals


def _gate_body(x_ref, w_ref, bias_ref, w_out_ref, i_out_ref, wt_ref):
    @pl.when(pl.program_id(0) == 0)
    def _():
        wt_ref[...] = w_ref[...].T

    wt = wt_ref[...]
    bias = bias_ref[...]
    scs = []
    for h in range(_NSUB):
        x = x_ref[h * _BS:(h + 1) * _BS, :]
        scs.append(jax.lax.dot_general(
            x, wt, (((1,), (0,)), ((), ())),
            preferred_element_type=jnp.float32))
    for h in range(_NSUB):
        s = jnp.sqrt(jax.nn.softplus(scs[h]))
        idxs, vals = _topk_rows(s, s + bias)
        inv = _SCALE / (vals[0] + vals[1] + vals[2]
                        + vals[3] + vals[4] + vals[5])
        col = pl.ds(h * _BS, _BS)
        for j in range(_K):
            i_out_ref[j, col] = idxs[j]
            w_out_ref[j, col] = vals[j] * inv


def kernel(x, weight, bias):
    bias2 = bias.reshape(1, _NE)
    w_out, i_out = pl.pallas_call(
        _gate_body,
        grid=(_TOKENS // _BT,),
        in_specs=[
            pl.BlockSpec((_BT, _DIM), lambda i: (i, 0)),
            pl.BlockSpec((_NE, _DIM), lambda i: (0, 0)),
            pl.BlockSpec((1, _NE), lambda i: (0, 0)),
        ],
        out_specs=[
            pl.BlockSpec((8, _BT), lambda i: (0, i)),
            pl.BlockSpec((8, _BT), lambda i: (0, i)),
        ],
        out_shape=[
            jax.ShapeDtypeStruct((8, _TOKENS), jnp.float32),
            jax.ShapeDtypeStruct((8, _TOKENS), jnp.float32),
        ],
        scratch_shapes=[pltpu.VMEM((_DIM, _NE), jnp.float32)],
    )(x, weight, bias2)
    return w_out[:_K].T, i_out[:_K].T.astype(jnp.int32)


# speedup vs baseline: 1.5729x; 1.1759x over previous
import jax
import jax.numpy as jnp
from jax.experimental import pallas as pl
from jax.experimental.pallas import tpu as pltpu

_TOKENS = 8192
_DIM = 7168
_DH = _DIM // 2
_NE = 384
_K = 6
_SCALE = 2.5
_BT = 512
_NSUB = 4
_BS = _BT // _NSUB


def _topk_rows(s, biased):
    iota = jax.lax.broadcasted_iota(
        jnp.int32, (_BS, _NE), 1).astype(jnp.float32)
    cur = biased
    vals, idxs = [], []
    for j in range(_K):
        m = jnp.max(cur, axis=1, keepdims=True)
        idx = jnp.min(jnp.where(cur == m, iota, jnp.float32(_NE)), axis=1)
        sel = iota == idx[:, None]
        vals.append(jnp.sum(jnp.where(sel, s, 0.0), axis=1))
        idxs.append(idx)
        if j + 1 < _K:
            cur = jnp.where(sel, -jnp.inf, cur)
    return idxs, vals


def _gate_body(xa_ref, xb_ref, w_ref, bias_ref, w_out_ref, i_out_ref,
               wta_ref, wtb_ref):
    @pl.when(pl.program_id(0) == 0)
    def _():
        wta_ref[...] = w_ref[:, :_DH].T
        wtb_ref[...] = w_ref[:, _DH:].T

    wta = wta_ref[...]
    wtb = wtb_ref[...]
    bias = bias_ref[...]
    dn = (((1,), (0,)), ((), ()))
    scs = []
    for h in range(_NSUB):
        row = slice(h * _BS, (h + 1) * _BS)
        sc = jax.lax.dot_general(xa_ref[row, :], wta, dn,
                                 preferred_element_type=jnp.float32)
        sc += jax.lax.dot_general(xb_ref[row, :], wtb, dn,
                                  preferred_element_type=jnp.float32)
        scs.append(sc)
    for h in range(_NSUB):
        s = jnp.sqrt(jax.nn.softplus(scs[h]))
        idxs, vals = _topk_rows(s, s + bias)
        inv = _SCALE / (vals[0] + vals[1] + vals[2]
                        + vals[3] + vals[4] + vals[5])
        col = pl.ds(h * _BS, _BS)
        for j in range(_K):
            i_out_ref[j, col] = idxs[j]
            w_out_ref[j, col] = vals[j] * inv


def kernel(x, weight, bias):
    bias2 = bias.reshape(1, _NE)
    w_out, i_out = pl.pallas_call(
        _gate_body,
        grid=(_TOKENS // _BT,),
        in_specs=[
            pl.BlockSpec((_BT, _DH), lambda i: (i, 0)),
            pl.BlockSpec((_BT, _DH), lambda i: (i, 1)),
            pl.BlockSpec((_NE, _DIM), lambda i: (0, 0)),
            pl.BlockSpec((1, _NE), lambda i: (0, 0)),
        ],
        out_specs=[
            pl.BlockSpec((8, _BT), lambda i: (0, i)),
            pl.BlockSpec((8, _BT), lambda i: (0, i)),
        ],
        out_shape=[
            jax.ShapeDtypeStruct((8, _TOKENS), jnp.float32),
            jax.ShapeDtypeStruct((8, _TOKENS), jnp.float32),
        ],
        scratch_shapes=[pltpu.VMEM((_DH, _NE), jnp.float32),
                        pltpu.VMEM((_DH, _NE), jnp.float32)],
    )(x, x, weight, bias2)
    return w_out[:_K].T, i_out[:_K].T.astype(jnp.int32)
